# baseline (speedup 1.0000x reference)
import jax
import jax.numpy as jnp
from jax import lax
from jax.experimental import pallas as pl
from jax.experimental.pallas import tpu as pltpu
from jax.experimental.pallas import tpu_sc as plsc

NTOK = 1000
B, L = 1024, 50
NC, NS = 2, 16
BPW = B // (NC * NS)
SB = 16


def _body(x_hbm, zeros_hbm, out_hbm, zbuf, s0, s1):
    cid = lax.axis_index("c")
    tid = lax.axis_index("s")
    b0 = (cid * NS + tid) * BPW

    pltpu.sync_copy(zeros_hbm, zbuf.at[pl.ds(tid, 1)])
    plsc.subcore_barrier()

    c0 = pltpu.async_copy(zbuf, out_hbm.at[pl.ds(b0, SB)], s0)
    c1 = pltpu.async_copy(zbuf, out_hbm.at[pl.ds(b0 + SB, SB)], s1)
    c0.wait()
    c1.wait()


@jax.jit
def kernel(x):
    mesh = plsc.VectorSubcoreMesh(
        core_axis_name="c", subcore_axis_name="s",
        num_cores=NC, num_subcores=NS,
    )
    run = pl.kernel(
        _body,
        out_type=jax.ShapeDtypeStruct((B, L, NTOK), jnp.float32),
        mesh=mesh,
        scratch_types=[
            pltpu.VMEM_SHARED((SB, L, NTOK), jnp.float32),
            pltpu.SemaphoreType.DMA,
            pltpu.SemaphoreType.DMA,
        ],
        compiler_params=pltpu.CompilerParams(
            needs_layout_passes=False,
            use_tc_tiling_on_sc=True,
        ),
    )
    zeros = jnp.zeros((1, L, NTOK), jnp.float32)
    return run(x.reshape(B * L).astype(jnp.int32), zeros)


# speedup vs baseline: 1.1372x; 1.1372x over previous
import jax
import jax.numpy as jnp
from jax import lax
from jax.experimental import pallas as pl
from jax.experimental.pallas import tpu as pltpu
from jax.experimental.pallas import tpu_sc as plsc

NTOK = 1000
B, L = 1024, 50
NC, NS = 2, 16
BPW = B // (NC * NS)
LANES = 16
LA = 48
VA = 896
GROUPS = (0, 16, 32, L - LANES)


def _body(x_hbm, out_hbm, xv,
          bufm0, bufm1, bufa0, bufa1, bufb0, bufb1, *sems):
    wid = lax.axis_index("c") * NS + lax.axis_index("s")
    b0 = wid * BPW
    bufm = (bufm0, bufm1)
    bufa = (bufa0, bufa1)
    bufb = (bufb0, bufb1)

    pltpu.sync_copy(x_hbm.at[pl.ds(b0 * L, BPW * L)], xv)

    z = jnp.zeros((LANES,), jnp.float32)

    def _zero(ref, rows, cols):
        def _row(r):
            for c in range(cols // LANES):
                ref[0, r, pl.ds(c * LANES, LANES)] = z
            if cols % LANES:
                ref[0, r, pl.ds(cols - LANES, LANES)] = z

        pl.loop(0, rows)(_row)

    for s in range(2):
        _zero(bufm[s], LA, VA)
        _zero(bufa[s], LA, NTOK - VA)
        _zero(bufb[s], L - LA, NTOK)

    lane = lax.iota(jnp.int32, LANES)
    ones = jnp.ones((LANES,), jnp.float32)
    zeros = jnp.zeros((LANES,), jnp.float32)
    zi = jnp.zeros((LANES,), jnp.int32)

    def scatter(s, i, value):
        for l0 in GROUPS:
            l = l0 + lane
            v = xv[pl.ds(i * L + l0, LANES)]
            ok = (v >= 0) & (v < NTOK)
            inm = ok & (l < LA) & (v < VA)
            ina = ok & (l < LA) & (v >= VA)
            inb = ok & (l >= LA)
            plsc.store_scatter(
                bufm[s], [zi, jnp.where(inm, l, 0), jnp.where(inm, v, 0)],
                value, mask=inm)
            plsc.store_scatter(
                bufa[s], [zi, jnp.where(ina, l, 0),
                          jnp.where(ina, v - VA, 0)],
                value, mask=ina)
            plsc.store_scatter(
                bufb[s], [zi, jnp.where(inb, l - LA, 0),
                          jnp.where(inb, v, 0)],
                value, mask=inb)

    copies = [None, None]
    for i in range(BPW):
        s = i % 2
        if copies[s] is not None:
            for c in copies[s]:
                c.wait()
            scatter(s, i - 2, zeros)
        scatter(s, i, ones)
        bb = b0 + i
        copies[s] = (
            pltpu.async_copy(
                bufm[s], out_hbm.at[pl.ds(bb, 1), pl.ds(0, LA),
                                    pl.ds(0, VA)], sems[3 * s]),
            pltpu.async_copy(
                bufa[s], out_hbm.at[pl.ds(bb, 1), pl.ds(0, LA),
                                    pl.ds(VA, NTOK - VA)], sems[3 * s + 1]),
            pltpu.async_copy(
                bufb[s], out_hbm.at[pl.ds(bb, 1), pl.ds(LA, L - LA),
                                    pl.ds(0, NTOK)], sems[3 * s + 2]),
        )
    for s in ((BPW - 1) % 2, BPW % 2):
        for c in copies[s]:
            c.wait()


@jax.jit
def kernel(x):
    mesh = plsc.VectorSubcoreMesh(
        core_axis_name="c", subcore_axis_name="s",
        num_cores=NC, num_subcores=NS,
    )
    run = pl.kernel(
        _body,
        out_type=jax.ShapeDtypeStruct((B, L, NTOK), jnp.float32),
        mesh=mesh,
        scratch_types=[
            pltpu.VMEM((BPW * L,), jnp.int32),
            pltpu.VMEM((1, LA, VA), jnp.float32),
            pltpu.VMEM((1, LA, VA), jnp.float32),
            pltpu.VMEM((1, LA, NTOK - VA), jnp.float32),
            pltpu.VMEM((1, LA, NTOK - VA), jnp.float32),
            pltpu.VMEM((1, L - LA, NTOK), jnp.float32),
            pltpu.VMEM((1, L - LA, NTOK), jnp.float32),
            pltpu.SemaphoreType.DMA,
            pltpu.SemaphoreType.DMA,
            pltpu.SemaphoreType.DMA,
            pltpu.SemaphoreType.DMA,
            pltpu.SemaphoreType.DMA,
            pltpu.SemaphoreType.DMA,
        ],
        compiler_params=pltpu.CompilerParams(
            needs_layout_passes=False,
            use_tc_tiling_on_sc=True,
        ),
    )
    return run(x.reshape(B * L).astype(jnp.int32))
